# Initial kernel scaffold; baseline (speedup 1.0000x reference)
import functools

import jax
import jax.numpy as jnp
import numpy as np
from jax import lax
from jax.experimental import pallas as pl
from jax.experimental.pallas import tpu as pltpu
from jax.experimental.pallas import tpu_sc as plsc

VOCAB = 1000000
EMBED = 64
BATCH = 16384
CTX = 20

NC = 2
NS = 16
LANES = 16
NW = NC * NS
B_PER_W = BATCH // NW
ROWS_PER_W = B_PER_W * CTX
CHUNK = 128
N_CHUNKS = ROWS_PER_W // CHUNK
CEN_CHUNKS = B_PER_W // CHUNK
TOTAL_ROWS = BATCH * CTX
GROUPS = CHUNK // LANES


def _sc_body(wi_hbm, wo_hbm, cen_hbm, ctx_hbm, neg_hbm, pos_out, neg_out,
             cen_idx_v, ctx_idx_v, neg_idx_v, wi_buf, rows_v, tmp_v,
             pos_sc_v, neg_sc_v, sem0, sem1):
  wid = lax.axis_index("s") * NC + lax.axis_index("c")
  sems = (sem0, sem1)

  pltpu.sync_copy(cen_hbm.at[pl.ds(wid * CEN_CHUNKS, CEN_CHUNKS)], cen_idx_v)
  pltpu.sync_copy(ctx_hbm.at[pl.ds(wid * N_CHUNKS, N_CHUNKS)], ctx_idx_v)
  pltpu.sync_copy(neg_hbm.at[pl.ds(wid * N_CHUNKS, N_CHUNKS)], neg_idx_v)

  for j in range(CEN_CHUNKS):
    pltpu.async_copy(wi_hbm.at[cen_idx_v.at[j]],
                     wi_buf.at[pl.ds(j * CHUNK, CHUNK)], sem0).wait()

  col_base = lax.iota(jnp.int32, LANES) * LANES

  def run_phase(idx_v, sc_v):
    def issue(c, par):
      pltpu.async_copy(wo_hbm.at[idx_v.at[c]], rows_v.at[par], sems[par])

    def wait(c, par):
      pltpu.make_async_copy(wo_hbm.at[idx_v.at[c]], rows_v.at[par],
                            sems[par]).wait()

    def compute(c, par):
      rows = rows_v.at[par]
      for grp in range(GROUPS):
        for i in range(LANES):
          r_local = grp * LANES + i
          r = c * CHUNK + r_local
          b_loc = lax.div(r, CTX)
          t = None
          for g in range(EMBED // LANES):
            wo_v = rows[r_local, pl.ds(g * LANES, LANES)]
            wi_v = wi_buf[b_loc, pl.ds(g * LANES, LANES)]
            p = wo_v * wi_v
            t = p if t is None else t + p
          tmp_v[pl.ds(i * LANES, LANES)] = t
        sv = None
        for cc in range(LANES):
          col = plsc.load_gather(tmp_v, [col_base + cc])
          sv = col if sv is None else sv + col
        sc_v[pl.ds(c * CHUNK + grp * LANES, LANES)] = sv

    issue(0, 0)
    issue(1, 1)

    def body(gi, carry):
      for par in range(2):
        c = gi * 2 + par
        wait(c, par)
        compute(c, par)
        nxt = c + 2

        @pl.when(nxt < N_CHUNKS)
        def _():
          issue(nxt, par)
      return carry

    lax.fori_loop(0, N_CHUNKS // 2, body, 0)

  run_phase(ctx_idx_v, pos_sc_v)
  run_phase(neg_idx_v, neg_sc_v)

  pltpu.sync_copy(pos_sc_v, pos_out.at[pl.ds(wid * ROWS_PER_W, ROWS_PER_W)])
  pltpu.sync_copy(neg_sc_v, neg_out.at[pl.ds(wid * ROWS_PER_W, ROWS_PER_W)])


_sc_scores = functools.partial(
    pl.kernel,
    out_type=[jax.ShapeDtypeStruct((TOTAL_ROWS,), jnp.float32)] * 2,
    mesh=plsc.VectorSubcoreMesh(core_axis_name="c", subcore_axis_name="s"),
    scratch_types=[
        pltpu.VMEM((CEN_CHUNKS, CHUNK), jnp.int32),
        pltpu.VMEM((N_CHUNKS, CHUNK), jnp.int32),
        pltpu.VMEM((N_CHUNKS, CHUNK), jnp.int32),
        pltpu.VMEM((B_PER_W, EMBED), jnp.float32),
        pltpu.VMEM((2, CHUNK, EMBED), jnp.float32),
        pltpu.VMEM((LANES * LANES,), jnp.float32),
        pltpu.VMEM((ROWS_PER_W,), jnp.float32),
        pltpu.VMEM((ROWS_PER_W,), jnp.float32),
        pltpu.SemaphoreType.DMA,
        pltpu.SemaphoreType.DMA,
    ],
)(_sc_body)


def _tc_body(p_ref, n_ref, o_ref):
  p = p_ref[...]
  n = n_ref[...]

  def logsig(x):
    return jnp.minimum(x, 0.0) - jnp.log1p(jnp.exp(-jnp.abs(x)))

  t = logsig(p) + logsig(-n)
  o_ref[0, 0] = -jnp.sum(t) / np.float32(BATCH * CTX)


def _tc_loss(pos2d, neg2d):
  return pl.pallas_call(
      _tc_body,
      out_shape=jax.ShapeDtypeStruct((1, 1), jnp.float32),
      out_specs=pl.BlockSpec(memory_space=pltpu.SMEM),
  )(pos2d, neg2d)


def kernel(Wi, Wo, center, contexts, neg_samples):
  cen = center.reshape(BATCH // CHUNK, CHUNK).astype(jnp.int32)
  ctx = contexts.reshape(TOTAL_ROWS // CHUNK, CHUNK).astype(jnp.int32)
  neg = neg_samples.reshape(TOTAL_ROWS // CHUNK, CHUNK).astype(jnp.int32)
  pos_sc, neg_sc = _sc_scores(Wi, Wo, cen, ctx, neg)
  out = _tc_loss(pos_sc.reshape(-1, CHUNK), neg_sc.reshape(-1, CHUNK))
  return out[0, 0]


# speedup vs baseline: 4.0471x; 4.0471x over previous
import functools

import jax
import jax.numpy as jnp
import numpy as np
from jax import lax
from jax.experimental import pallas as pl
from jax.experimental.pallas import tpu as pltpu
from jax.experimental.pallas import tpu_sc as plsc

VOCAB = 1000000
EMBED = 64
BATCH = 16384
CTX = 20

NC = 2
NS = 16
LANES = 16
NW = NC * NS
B_PER_W = BATCH // NW
ROWS_PER_W = B_PER_W * CTX
CHUNK = 128
N_CHUNKS = ROWS_PER_W // CHUNK
CEN_CHUNKS = B_PER_W // CHUNK
TOTAL_ROWS = BATCH * CTX
GROUPS = CHUNK // LANES


def _sc_body(wi_hbm, wo_hbm, cen_hbm, ctx_hbm, neg_hbm, pos_out, neg_out,
             cen_idx_v, ctx_idx_v, neg_idx_v, wi_buf, rows_v, tmp_v,
             pos_sc_v, neg_sc_v, sem0, sem1):
  wid = lax.axis_index("s") * NC + lax.axis_index("c")
  sems = (sem0, sem1)

  pltpu.sync_copy(cen_hbm.at[pl.ds(wid * CEN_CHUNKS, CEN_CHUNKS)], cen_idx_v)
  pltpu.sync_copy(ctx_hbm.at[pl.ds(wid * N_CHUNKS, N_CHUNKS)], ctx_idx_v)
  pltpu.sync_copy(neg_hbm.at[pl.ds(wid * N_CHUNKS, N_CHUNKS)], neg_idx_v)

  for j in range(CEN_CHUNKS):
    pltpu.async_copy(wi_hbm.at[cen_idx_v.at[j]],
                     wi_buf.at[pl.ds(j * CHUNK, CHUNK)], sem0).wait()

  col_base = lax.iota(jnp.int32, LANES) * LANES

  def run_phase(idx_v, sc_v):
    def issue(c, par):
      pltpu.async_copy(wo_hbm.at[idx_v.at[c]], rows_v.at[par], sems[par])

    def wait(c, par):
      pltpu.make_async_copy(wo_hbm.at[idx_v.at[c]], rows_v.at[par],
                            sems[par]).wait()

    def compute(c, par):
      rows = rows_v.at[par]
      for grp in range(GROUPS):
        for i in range(LANES):
          r_local = grp * LANES + i
          r = c * CHUNK + r_local
          b_loc = lax.div(r, CTX)
          t = None
          for g in range(EMBED // LANES):
            wo_v = rows[r_local, pl.ds(g * LANES, LANES)]
            wi_v = wi_buf[b_loc, pl.ds(g * LANES, LANES)]
            p = wo_v * wi_v
            t = p if t is None else t + p
          tmp_v[pl.ds(i * LANES, LANES)] = t
        sv = None
        for cc in range(LANES):
          col = plsc.load_gather(tmp_v, [col_base + cc])
          sv = col if sv is None else sv + col
        sc_v[pl.ds(c * CHUNK + grp * LANES, LANES)] = sv

    issue(0, 0)
    issue(1, 1)

    def body(gi, carry):
      for par in range(2):
        c = gi * 2 + par
        wait(c, par)
        compute(c, par)
        nxt = c + 2

        @pl.when(nxt < N_CHUNKS)
        def _():
          issue(nxt, par)
      return carry

    lax.fori_loop(0, N_CHUNKS // 2, body, 0)

  run_phase(ctx_idx_v, pos_sc_v)
  run_phase(neg_idx_v, neg_sc_v)

  pltpu.sync_copy(pos_sc_v, pos_out.at[pl.ds(wid * ROWS_PER_W, ROWS_PER_W)])
  pltpu.sync_copy(neg_sc_v, neg_out.at[pl.ds(wid * ROWS_PER_W, ROWS_PER_W)])


_sc_scores = functools.partial(
    pl.kernel,
    out_type=[jax.ShapeDtypeStruct((TOTAL_ROWS,), jnp.float32)] * 2,
    mesh=plsc.VectorSubcoreMesh(core_axis_name="c", subcore_axis_name="s"),
    compiler_params=pltpu.CompilerParams(
        needs_layout_passes=False, use_tc_tiling_on_sc=False),
    scratch_types=[
        pltpu.VMEM((CEN_CHUNKS, CHUNK), jnp.int32),
        pltpu.VMEM((N_CHUNKS, CHUNK), jnp.int32),
        pltpu.VMEM((N_CHUNKS, CHUNK), jnp.int32),
        pltpu.VMEM((B_PER_W, EMBED), jnp.float32),
        pltpu.VMEM((2, CHUNK, EMBED), jnp.float32),
        pltpu.VMEM((LANES * LANES,), jnp.float32),
        pltpu.VMEM((ROWS_PER_W,), jnp.float32),
        pltpu.VMEM((ROWS_PER_W,), jnp.float32),
        pltpu.SemaphoreType.DMA,
        pltpu.SemaphoreType.DMA,
    ],
)(_sc_body)


def _tc_body(p_ref, n_ref, o_ref):
  p = p_ref[...]
  n = n_ref[...]

  def logsig(x):
    return jnp.minimum(x, 0.0) - jnp.log1p(jnp.exp(-jnp.abs(x)))

  t = logsig(p) + logsig(-n)
  o_ref[0, 0] = -jnp.sum(t) / np.float32(BATCH * CTX)


def _tc_loss(pos2d, neg2d):
  return pl.pallas_call(
      _tc_body,
      out_shape=jax.ShapeDtypeStruct((1, 1), jnp.float32),
      out_specs=pl.BlockSpec(memory_space=pltpu.SMEM),
  )(pos2d, neg2d)


def kernel(Wi, Wo, center, contexts, neg_samples):
  cen = center.reshape(BATCH // CHUNK, CHUNK).astype(jnp.int32)
  ctx = contexts.reshape(TOTAL_ROWS // CHUNK, CHUNK).astype(jnp.int32)
  neg = neg_samples.reshape(TOTAL_ROWS // CHUNK, CHUNK).astype(jnp.int32)
  pos_sc, neg_sc = _sc_scores(Wi, Wo, cen, ctx, neg)
  out = _tc_loss(pos_sc.reshape(-1, CHUNK), neg_sc.reshape(-1, CHUNK))
  return out[0, 0]
